# baseline (speedup 1.0000x reference)
import functools

import jax
import jax.numpy as jnp
from jax import lax
from jax.experimental import pallas as pl
from jax.experimental.pallas import tpu as pltpu
from jax.experimental.pallas import tpu_sc as plsc

HIDDEN = 1024
L = 16
NCH = HIDDEN // L
NC, NS = 2, 16
NW = NC * NS
EPS = 1e-12
C = 32


_GATHER_DN = lax.GatherDimensionNumbers(
    offset_dims=(), collapsed_slice_dims=(0,), start_index_map=(0,)
)


def _lane_shuffle(v, idx):
    return lax.gather(
        v, idx[:, None], _GATHER_DN, slice_sizes=(1,),
        mode=lax.GatherScatterMode.PROMISE_IN_BOUNDS,
    )


def _xlane_sum(v):
    idx = lax.iota(jnp.int32, L)
    for k in (8, 4, 2, 1):
        v = v + _lane_shuffle(v, idx ^ k)
    return v


def _ln_rows(x_v, pos_v, gamma_v, beta_v, n_rows):
    def tok_body(t, _):
        def p1(j, carry):
            s, ss = carry
            sl = pl.ds(j * L, L)
            v = x_v[t, sl] + pos_v[t, sl]
            x_v[t, sl] = v
            return s + v, ss + v * v

        zero = jnp.zeros((L,), jnp.float32)
        s, ss = lax.fori_loop(0, NCH, p1, (zero, zero))
        mean = _xlane_sum(s) * (1.0 / HIDDEN)
        var = _xlane_sum(ss) * (1.0 / HIDDEN) - mean * mean
        xv = var + EPS
        i = lax.bitcast_convert_type(xv, jnp.int32)
        i = 0x5F3759DF - lax.shift_right_logical(i, 1)
        y = lax.bitcast_convert_type(i, jnp.float32)
        for _ in range(3):
            y = y * (1.5 - 0.5 * xv * y * y)
        a = y
        b = (-mean) * y

        def p2(j, _):
            sl = pl.ds(j * L, L)
            v = x_v[t, sl]
            x_v[t, sl] = (v * a + b) * gamma_v[sl] + beta_v[sl]
            return 0

        lax.fori_loop(0, NCH, p2, 0)
        return 0

    lax.fori_loop(0, n_rows, tok_body, 0)


def _make_sc_kernel(tokens, seq):
    tpw = tokens // NW
    n_chunks = tpw // C

    mesh = plsc.VectorSubcoreMesh(
        core_axis_name="c", subcore_axis_name="s", num_cores=NC, num_subcores=NS
    )

    @functools.partial(
        pl.kernel,
        out_type=jax.ShapeDtypeStruct((tokens, HIDDEN), jnp.float32),
        mesh=mesh,
        scratch_types=[
            pltpu.VMEM((C,), jnp.int32),
            pltpu.VMEM((C, HIDDEN), jnp.float32),
            pltpu.VMEM((C, HIDDEN), jnp.float32),
            pltpu.VMEM((HIDDEN,), jnp.float32),
            pltpu.VMEM((HIDDEN,), jnp.float32),
            pltpu.SemaphoreType.DMA,
        ],
    )
    def emb_kernel(ids_hbm, word_hbm, pos_hbm, gamma_hbm, beta_hbm, out_hbm,
                   idx_v, x_v, pos_v, gamma_v, beta_v, sem):
        wid = lax.axis_index("s") * NC + lax.axis_index("c")
        base_w = wid * tpw
        s0 = lax.rem(base_w, seq)
        pltpu.sync_copy(gamma_hbm, gamma_v)
        pltpu.sync_copy(beta_hbm, beta_v)
        for g in range(n_chunks):
            base = base_w + g * C
            pltpu.sync_copy(ids_hbm.at[pl.ds(base, C)], idx_v)
            pltpu.sync_copy(pos_hbm.at[pl.ds(s0 + g * C, C)], pos_v)
            pltpu.async_copy(word_hbm.at[idx_v], x_v, sem).wait()
            _ln_rows(x_v, pos_v, gamma_v, beta_v, C)
            pltpu.sync_copy(x_v, out_hbm.at[pl.ds(base, C)])

    return emb_kernel


def kernel(input_ids, word_table, pos_table, ln_gamma, ln_beta):
    batch, seq = input_ids.shape
    tokens = batch * seq
    ids = input_ids.reshape(tokens).astype(jnp.int32)
    emb = _make_sc_kernel(tokens, seq)
    out = emb(ids, word_table, pos_table, ln_gamma, ln_beta)
    return out.reshape(batch, seq, HIDDEN)


# speedup vs baseline: 1.0535x; 1.0535x over previous
import functools

import jax
import jax.numpy as jnp
from jax import lax
from jax.experimental import pallas as pl
from jax.experimental.pallas import tpu as pltpu
from jax.experimental.pallas import tpu_sc as plsc

HIDDEN = 1024
L = 16
NCH = HIDDEN // L
NC, NS = 2, 16
NW = NC * NS
EPS = 1e-12
C = 32


_GATHER_DN = lax.GatherDimensionNumbers(
    offset_dims=(), collapsed_slice_dims=(0,), start_index_map=(0,)
)


def _lane_shuffle(v, idx):
    return lax.gather(
        v, idx[:, None], _GATHER_DN, slice_sizes=(1,),
        mode=lax.GatherScatterMode.PROMISE_IN_BOUNDS,
    )


def _xlane_sum(v):
    idx = lax.iota(jnp.int32, L)
    for k in (8, 4, 2, 1):
        v = v + _lane_shuffle(v, idx ^ k)
    return v


def _ln_rows(x_v, pos_v, gamma_v, beta_v, n_rows):
    UNROLL = 8

    def tok_body(t, _):
        def p1(j, carry):
            s, ss = carry
            vs = []
            for k in range(UNROLL):
                sl = pl.ds((j * UNROLL + k) * L, L)
                v = x_v[t, sl] + pos_v[t, sl]
                x_v[t, sl] = v
                vs.append(v)
            sq = [v * v for v in vs]
            while len(vs) > 1:
                vs = [vs[i] + vs[i + 1] for i in range(0, len(vs), 2)]
                sq = [sq[i] + sq[i + 1] for i in range(0, len(sq), 2)]
            return s + vs[0], ss + sq[0]

        zero = jnp.zeros((L,), jnp.float32)
        s, ss = lax.fori_loop(0, NCH // UNROLL, p1, (zero, zero))
        mean = _xlane_sum(s) * (1.0 / HIDDEN)
        var = _xlane_sum(ss) * (1.0 / HIDDEN) - mean * mean
        xv = var + EPS
        i = lax.bitcast_convert_type(xv, jnp.int32)
        i = 0x5F3759DF - lax.shift_right_logical(i, 1)
        y = lax.bitcast_convert_type(i, jnp.float32)
        for _ in range(3):
            y = y * (1.5 - 0.5 * xv * y * y)
        a = y
        b = (-mean) * y

        def p2(j, _):
            for k in range(UNROLL):
                sl = pl.ds((j * UNROLL + k) * L, L)
                v = x_v[t, sl]
                x_v[t, sl] = (v * a + b) * gamma_v[sl] + beta_v[sl]
            return 0

        lax.fori_loop(0, NCH // UNROLL, p2, 0)
        return 0

    lax.fori_loop(0, n_rows, tok_body, 0)


def _make_sc_kernel(tokens, seq):
    tpw = tokens // NW
    n_chunks = tpw // C

    mesh = plsc.VectorSubcoreMesh(
        core_axis_name="c", subcore_axis_name="s", num_cores=NC, num_subcores=NS
    )

    @functools.partial(
        pl.kernel,
        out_type=jax.ShapeDtypeStruct((tokens, HIDDEN), jnp.float32),
        mesh=mesh,
        scratch_types=[
            pltpu.VMEM((C,), jnp.int32),
            pltpu.VMEM((C, HIDDEN), jnp.float32),
            pltpu.VMEM((C, HIDDEN), jnp.float32),
            pltpu.VMEM((HIDDEN,), jnp.float32),
            pltpu.VMEM((HIDDEN,), jnp.float32),
            pltpu.SemaphoreType.DMA,
        ],
    )
    def emb_kernel(ids_hbm, word_hbm, pos_hbm, gamma_hbm, beta_hbm, out_hbm,
                   idx_v, x_v, pos_v, gamma_v, beta_v, sem):
        wid = lax.axis_index("s") * NC + lax.axis_index("c")
        base_w = wid * tpw
        s0 = lax.rem(base_w, seq)
        pltpu.sync_copy(gamma_hbm, gamma_v)
        pltpu.sync_copy(beta_hbm, beta_v)
        for g in range(n_chunks):
            base = base_w + g * C
            pltpu.sync_copy(ids_hbm.at[pl.ds(base, C)], idx_v)
            pltpu.sync_copy(pos_hbm.at[pl.ds(s0 + g * C, C)], pos_v)
            pltpu.async_copy(word_hbm.at[idx_v], x_v, sem).wait()
            _ln_rows(x_v, pos_v, gamma_v, beta_v, C)
            pltpu.sync_copy(x_v, out_hbm.at[pl.ds(base, C)])

    return emb_kernel


def kernel(input_ids, word_table, pos_table, ln_gamma, ln_beta):
    batch, seq = input_ids.shape
    tokens = batch * seq
    ids = input_ids.reshape(tokens).astype(jnp.int32)
    emb = _make_sc_kernel(tokens, seq)
    out = emb(ids, word_table, pos_table, ln_gamma, ln_beta)
    return out.reshape(batch, seq, HIDDEN)
